# baseline (speedup 1.0000x reference)
import functools

import jax
import jax.numpy as jnp
from jax import lax
from jax.experimental import pallas as pl
from jax.experimental.pallas import tpu as pltpu
from jax.experimental.pallas import tpu_sc as plsc

NUM_ROWS = 8
DIM = 100
DIM_PAD = 128

_info = plsc.get_sparse_core_info()
_NC, _NS = _info.num_cores, _info.num_subcores
_NW = _NC * _NS


def _make_sc_gather(B: int, C: int):
    per_w = B // _NW
    n_chunks = per_w // C
    mesh = plsc.VectorSubcoreMesh(core_axis_name="c", subcore_axis_name="s")

    @functools.partial(
        pl.kernel,
        mesh=mesh,
        out_type=jax.ShapeDtypeStruct((B, DIM_PAD), jnp.float32),
        scratch_types=[
            pltpu.VMEM((C,), jnp.int32),
            pltpu.VMEM((C, DIM_PAD), jnp.float32),
            pltpu.SemaphoreType.DMA,
        ],
    )
    def k(idx_hbm, table_hbm, out_hbm, idx_v, rows_v, sem):
        wid = lax.axis_index("s") * _NC + lax.axis_index("c")

        def step(g, carry):
            base = wid * per_w + g * C
            pltpu.sync_copy(idx_hbm.at[pl.ds(base, C)], idx_v)
            pltpu.async_copy(table_hbm.at[idx_v], rows_v, sem).wait()
            pltpu.sync_copy(rows_v, out_hbm.at[pl.ds(base, C)])
            return carry

        lax.fori_loop(0, n_chunks, step, 0)

    return k


def kernel(input, table):
    idx = input.reshape(-1).astype(jnp.int32)
    table_pad = jnp.pad(table, ((0, 0), (0, DIM_PAD - DIM)))
    out = _make_sc_gather(idx.shape[0], 512)(idx, table_pad)
    return out[:, :DIM].reshape(input.shape + (DIM,))


# speedup vs baseline: 1.0026x; 1.0026x over previous
import functools

import jax
import jax.numpy as jnp
from jax import lax
from jax.experimental import pallas as pl
from jax.experimental.pallas import tpu as pltpu
from jax.experimental.pallas import tpu_sc as plsc

NUM_ROWS = 8
DIM = 100
DIM_PAD = 128

_info = plsc.get_sparse_core_info()
_NC, _NS = _info.num_cores, _info.num_subcores
_NW = _NC * _NS


def _make_sc_gather(B: int, C: int):
    per_w = B // _NW
    n_chunks = per_w // C
    assert n_chunks % 2 == 0
    mesh = plsc.VectorSubcoreMesh(core_axis_name="c", subcore_axis_name="s")

    @functools.partial(
        pl.kernel,
        mesh=mesh,
        out_type=jax.ShapeDtypeStruct((B, DIM_PAD), jnp.float32),
        scratch_types=[
            pltpu.VMEM((per_w,), jnp.int32),
            pltpu.VMEM((C, DIM_PAD), jnp.float32),
            pltpu.VMEM((C, DIM_PAD), jnp.float32),
            pltpu.SemaphoreType.DMA,
            pltpu.SemaphoreType.DMA,
            pltpu.SemaphoreType.DMA,
            pltpu.SemaphoreType.DMA,
        ],
    )
    def k(idx_hbm, table_hbm, out_hbm, idx_v, rows0, rows1,
          sg0, sg1, sw0, sw1):
        wid = lax.axis_index("s") * _NC + lax.axis_index("c")
        w0 = wid * per_w
        rows = (rows0, rows1)
        sg = (sg0, sg1)
        sw = (sw0, sw1)

        pltpu.sync_copy(idx_hbm.at[pl.ds(w0, per_w)], idx_v)

        def gather_start(g, b):
            pltpu.async_copy(
                table_hbm.at[idx_v.at[pl.ds(g * C, C)]], rows[b], sg[b])

        def gather_wait(g, b):
            pltpu.make_async_copy(
                table_hbm.at[idx_v.at[pl.ds(g * C, C)]], rows[b], sg[b]
            ).wait()

        def wout_start(g, b):
            pltpu.async_copy(rows[b], out_hbm.at[pl.ds(w0 + g * C, C)], sw[b])

        def wout_wait(g, b):
            pltpu.make_async_copy(
                rows[b], out_hbm.at[pl.ds(w0 + g * C, C)], sw[b]).wait()

        gather_start(0, 0)
        gather_start(1, 1)

        def body(i, carry):
            for b in (0, 1):
                g = 2 * i + b
                gather_wait(g, b)
                wout_start(g, b)
                @pl.when(i < n_chunks // 2 - 1)
                def _():
                    wout_wait(g, b)
                    gather_start(g + 2, b)
            return carry

        lax.fori_loop(0, n_chunks // 2, body, 0)
        wout_wait(n_chunks - 2, 0)
        wout_wait(n_chunks - 1, 1)

    return k


def kernel(input, table):
    idx = input.reshape(-1).astype(jnp.int32)
    table_pad = jnp.pad(table, ((0, 0), (0, DIM_PAD - DIM)))
    out = _make_sc_gather(idx.shape[0], 320)(idx, table_pad)
    return out[:, :DIM].reshape(input.shape + (DIM,))


# speedup vs baseline: 3.9333x; 3.9231x over previous
import functools

import jax
import jax.numpy as jnp
from jax import lax
from jax.experimental import pallas as pl
from jax.experimental.pallas import tpu as pltpu
from jax.experimental.pallas import tpu_sc as plsc

NUM_ROWS = 8
DIM = 100
DIM_PAD = 128

_info = plsc.get_sparse_core_info()
_NC, _NS = _info.num_cores, _info.num_subcores
_NW = _NC * _NS


def _make_sc_gather(B: int, C: int):
    per_w = B // _NW
    n_chunks = per_w // C
    assert n_chunks % 2 == 0
    mesh = plsc.VectorSubcoreMesh(core_axis_name="c", subcore_axis_name="s")

    @functools.partial(
        pl.kernel,
        mesh=mesh,
        out_type=jax.ShapeDtypeStruct((B, DIM_PAD), jnp.float32),
        scratch_types=[
            pltpu.VMEM((per_w,), jnp.int32),
            pltpu.VMEM((C, DIM_PAD), jnp.float32),
            pltpu.VMEM((C, DIM_PAD), jnp.float32),
            pltpu.SemaphoreType.DMA,
            pltpu.SemaphoreType.DMA,
            pltpu.SemaphoreType.DMA,
            pltpu.SemaphoreType.DMA,
        ],
    )
    def k(idx_hbm, table_hbm, out_hbm, idx_v, rows0, rows1,
          sg0, sg1, sw0, sw1):
        wid = lax.axis_index("s") * _NC + lax.axis_index("c")
        w0 = wid * per_w
        rows = (rows0, rows1)
        sg = (sg0, sg1)
        sw = (sw0, sw1)

        pltpu.sync_copy(idx_hbm.at[pl.ds(w0, per_w)], idx_v)

        def gather_start(g, b):
            pltpu.async_copy(
                table_hbm.at[idx_v.at[pl.ds(g * C, C)]], rows[b], sg[b])

        def gather_wait(g, b):
            pltpu.make_async_copy(
                table_hbm.at[idx_v.at[pl.ds(g * C, C)]], rows[b], sg[b]
            ).wait()

        def wout_start(g, b):
            pltpu.async_copy(rows[b], out_hbm.at[pl.ds(w0 + g * C, C)], sw[b])

        def wout_wait(g, b):
            pltpu.make_async_copy(
                rows[b], out_hbm.at[pl.ds(w0 + g * C, C)], sw[b]).wait()

        gather_start(0, 0)
        gather_start(1, 1)

        def body(i, carry):
            for b in (0, 1):
                g = 2 * i + b
                gather_wait(g, b)
                wout_start(g, b)
                @pl.when(i < n_chunks // 2 - 1)
                def _():
                    wout_wait(g, b)
                    gather_start(g + 2, b)
            return carry

        lax.fori_loop(0, n_chunks // 2, body, 0)
        wout_wait(n_chunks - 2, 0)
        wout_wait(n_chunks - 1, 1)

    return k


REPLICAS = 64


def kernel(input, table):
    idx = input.reshape(-1).astype(jnp.int32)
    table_pad = jnp.pad(table, ((0, 0), (0, DIM_PAD - DIM)))
    table_rep = jnp.tile(table_pad, (REPLICAS, 1))
    idx = idx + NUM_ROWS * (
        jnp.arange(idx.shape[0], dtype=jnp.int32) % REPLICAS)
    out = _make_sc_gather(idx.shape[0], 320)(idx, table_rep)
    return out[:, :DIM].reshape(input.shape + (DIM,))
